# baseline (speedup 1.0000x reference)
import functools

import jax
import jax.numpy as jnp
from jax import lax
from jax.experimental import pallas as pl
from jax.experimental.pallas import tpu as pltpu
from jax.experimental.pallas import tpu_sc as plsc

VOCAB = 1000
VPAD = 1024
HIDDEN = 128
BATCH = 1024
SEQ = 50
N = BATCH * SEQ
NW = 32
BPW = N // NW
CHUNK = 80
NCHUNK = BPW // CHUNK


def _logit_table_body(emb_ref, w_ref, b_ref, m_ref):
    m_ref[...] = lax.dot_general(
        emb_ref[...], w_ref[...],
        dimension_numbers=(((1,), (1,)), ((), ())),
        preferred_element_type=jnp.float32,
    ) + b_ref[...]


def _logit_table(emb_table, proj_w_pad, proj_b_pad):
    return pl.pallas_call(
        _logit_table_body,
        out_shape=jax.ShapeDtypeStruct((VOCAB, VPAD), jnp.float32),
    )(emb_table, proj_w_pad, proj_b_pad)


@functools.cache
def _gather_logits():
    mesh = plsc.VectorSubcoreMesh(core_axis_name="c", subcore_axis_name="s")

    @functools.partial(
        pl.kernel,
        mesh=mesh,
        out_type=jax.ShapeDtypeStruct((N, VPAD), jnp.float32),
        scratch_types=[
            pltpu.VMEM((BPW,), jnp.int32),
            pltpu.VMEM((CHUNK, VPAD), jnp.float32),
            pltpu.SemaphoreType.DMA,
        ],
    )
    def gather(m_hbm, idx_hbm, out_hbm, idx_v, rows_v, sem):
        wid = lax.axis_index("s") * 2 + lax.axis_index("c")
        base = wid * BPW
        pltpu.sync_copy(idx_hbm.at[pl.ds(base, BPW)], idx_v)

        def body(g, carry):
            pltpu.async_copy(
                m_hbm.at[idx_v.at[pl.ds(g * CHUNK, CHUNK)]], rows_v, sem
            ).wait()
            pltpu.sync_copy(rows_v, out_hbm.at[pl.ds(base + g * CHUNK, CHUNK)])
            return carry

        lax.fori_loop(0, NCHUNK, body, 0)

    return gather


def kernel(input_ids, emb_table, proj_w, proj_b):
    w_pad = jnp.pad(proj_w, ((0, VPAD - VOCAB), (0, 0)))
    b_pad = jnp.pad(proj_b, (0, VPAD - VOCAB)).reshape(1, VPAD)
    m = _logit_table(emb_table, w_pad, b_pad)
    ids = input_ids.reshape(-1).astype(jnp.int32)
    out = _gather_logits()(m, ids)
    return out[:, :VOCAB].reshape(BATCH, SEQ, VOCAB)
